# baseline (speedup 1.0000x reference)
import functools

import jax
import jax.numpy as jnp
from jax import lax
from jax.experimental import pallas as pl
from jax.experimental.pallas import tpu as pltpu
from jax.experimental.pallas import tpu_sc as plsc

N = 10000
E = 320000
D = 128
EPS = 1e-5

NC = 2
NS = 16
NW = NC * NS
CHUNK = 128
NCHUNKS = E // CHUNK
G = 16
NGRP = 5
NCHP = 2560
WCH = NCHP // NW
NBUF = 4
NPAD = 10240
TROWS = NPAD // NS
ZB = 32
NPH = 5000
APAD = 5376
ATR = APAD // NS
AZB = 24

BN = 1000
NB = N // BN

_mesh = plsc.VectorSubcoreMesh(core_axis_name="c", subcore_axis_name="s")


def _leaky(x):
    return jnp.where(x >= 0, x, 0.01 * x)


@functools.partial(
    pl.kernel,
    out_type=jax.ShapeDtypeStruct((NC, 2, NPAD), jnp.float32),
    mesh=_mesh,
    scratch_types=[
        pltpu.VMEM((G, CHUNK), jnp.int32),
        pltpu.VMEM((G, CHUNK), jnp.int32),
        pltpu.VMEM((G, CHUNK), jnp.float32),
        pltpu.VMEM((TROWS,), jnp.float32),
        pltpu.VMEM_SHARED((NPAD,), jnp.float32),
        pltpu.VMEM_SHARED((NPAD,), jnp.float32),
    ],
)
def _deg(src_hbm, dst_hbm, out_hbm, sbuf, dbuf, ones_v, zvec, acc_s, acc_d):
    cid = lax.axis_index("c")
    sid = lax.axis_index("s")
    w = sid * NC + cid

    def fill_ones(j, carry):
        for k in range(CHUNK // 16):
            ones_v[j, pl.ds(k * 16, 16)] = jnp.ones((16,), jnp.float32)
        return carry

    lax.fori_loop(0, G, fill_ones, 0)

    def fill_z(q, carry):
        zvec[pl.ds(q * 16, 16)] = jnp.zeros((16,), jnp.float32)
        return carry

    lax.fori_loop(0, TROWS // 16, fill_z, 0)
    pltpu.sync_copy(zvec, acc_s.at[pl.ds(sid * TROWS, TROWS)])
    pltpu.sync_copy(zvec, acc_d.at[pl.ds(sid * TROWS, TROWS)])
    plsc.subcore_barrier()

    cstart = w * WCH

    def grp(g, carry):
        gs = cstart + g * G
        pltpu.sync_copy(src_hbm.at[pl.ds(gs, G)], sbuf)
        pltpu.sync_copy(dst_hbm.at[pl.ds(gs, G)], dbuf)

        def chunk(j, c2):
            pltpu.sync_copy(ones_v.at[0], acc_s.at[sbuf.at[j]], add=True)
            pltpu.sync_copy(ones_v.at[0], acc_d.at[dbuf.at[j]], add=True)
            return c2

        lax.fori_loop(0, G, chunk, 0)
        return carry

    lax.fori_loop(0, NGRP, grp, 0)
    plsc.subcore_barrier()
    pltpu.sync_copy(acc_s.at[pl.ds(sid * TROWS, TROWS)],
                    out_hbm.at[cid, 0, pl.ds(sid * TROWS, TROWS)])
    pltpu.sync_copy(acc_d.at[pl.ds(sid * TROWS, TROWS)],
                    out_hbm.at[cid, 1, pl.ds(sid * TROWS, TROWS)])


@functools.partial(
    pl.kernel,
    out_type=jax.ShapeDtypeStruct((NC, APAD, D), jnp.float32),
    mesh=_mesh,
    scratch_types=[
        pltpu.VMEM((2, G, CHUNK), jnp.int32),
        pltpu.VMEM((2, G, CHUNK), jnp.int32),
        pltpu.VMEM((CHUNK, D), jnp.float32),
        pltpu.VMEM((CHUNK, D), jnp.float32),
        pltpu.VMEM((CHUNK, D), jnp.float32),
        pltpu.VMEM((CHUNK, D), jnp.float32),
        pltpu.VMEM((AZB, D), jnp.float32),
        pltpu.VMEM_SHARED((APAD, D), jnp.float32),
        pltpu.SemaphoreType.DMA,
        pltpu.SemaphoreType.DMA,
        pltpu.SemaphoreType.DMA,
        pltpu.SemaphoreType.DMA,
        pltpu.SemaphoreType.DMA,
        pltpu.SemaphoreType.DMA,
        pltpu.SemaphoreType.DMA,
        pltpu.SemaphoreType.DMA,
    ],
)
def _segsum(h_hbm, src_hbm, dst_hbm, out_hbm,
            sbuf, dbuf, rows0, rows1, rows2, rows3, zbuf, acc,
            gs0, gs1, gs2, gs3, ss0, ss1, ss2, ss3):
    rows = (rows0, rows1, rows2, rows3)
    gsem = (gs0, gs1, gs2, gs3)
    ssem = (ss0, ss1, ss2, ss3)
    cid = lax.axis_index("c")
    sid = lax.axis_index("s")
    w = sid * NC + cid

    def zrow(i, carry):
        for k in range(D // 16):
            zbuf[i, pl.ds(k * 16, 16)] = jnp.zeros((16,), jnp.float32)
        return carry

    lax.fori_loop(0, AZB, zrow, 0)
    for b in range(ATR // AZB):
        pltpu.sync_copy(zbuf, acc.at[pl.ds(sid * ATR + b * AZB, AZB)])
    plsc.subcore_barrier()

    cstart = w * WCH

    def grp(g, carry):
        p = g % 2
        gs = cstart + g * G
        pltpu.sync_copy(src_hbm.at[pl.ds(gs, G)], sbuf.at[p])
        pltpu.sync_copy(dst_hbm.at[pl.ds(gs, G)], dbuf.at[p])

        def quad(kk, c2):
            for b in range(NBUF):
                j = kk * NBUF + b
                c = g * G + j

                @pl.when(c >= NBUF)
                def _():
                    pltpu.make_async_copy(
                        rows[b], acc.at[dbuf.at[p, j]], ssem[b]).wait()

                pltpu.async_copy(h_hbm.at[sbuf.at[p, j]], rows[b], gsem[b])

            for b in range(NBUF):
                j = kk * NBUF + b
                pltpu.make_async_copy(
                    h_hbm.at[sbuf.at[p, j]], rows[b], gsem[b]).wait()
                pltpu.async_copy(
                    rows[b], acc.at[dbuf.at[p, j]], ssem[b], add=True)

            return c2

        lax.fori_loop(0, G // NBUF, quad, 0)
        return carry

    lax.fori_loop(0, NGRP, grp, 0)
    for b in range(NBUF):
        pltpu.make_async_copy(rows[b], acc.at[dbuf.at[0, 0]], ssem[b]).wait()
    plsc.subcore_barrier()
    base = sid * ATR
    for r0, nr in ((0, 128), (128, 128), (256, 80)):
        pltpu.sync_copy(acc.at[pl.ds(base + r0, nr)],
                        out_hbm.at[cid, pl.ds(base + r0, nr)])


def _pre_body(x_ref, dp_ref, h0_ref, ns_ref, nd_ref):
    dp = dp_ref[...]
    ns = lax.rsqrt(jnp.maximum(dp[:, 0:1] + dp[:, 2:3], 1.0))
    nd = lax.rsqrt(jnp.maximum(dp[:, 1:2] + dp[:, 3:4], 1.0))
    ns_ref[...] = ns
    nd_ref[...] = nd
    h0_ref[...] = x_ref[...] * ns


def _pre(x, dp):
    return pl.pallas_call(
        _pre_body,
        grid=(NB,),
        in_specs=[
            pl.BlockSpec((BN, D), lambda i: (i, 0)),
            pl.BlockSpec((BN, 4), lambda i: (i, 0)),
        ],
        out_specs=[
            pl.BlockSpec((BN, D), lambda i: (i, 0)),
            pl.BlockSpec((BN, 1), lambda i: (i, 0)),
            pl.BlockSpec((BN, 1), lambda i: (i, 0)),
        ],
        out_shape=[
            jax.ShapeDtypeStruct((N, D), jnp.float32),
            jax.ShapeDtypeStruct((N, 1), jnp.float32),
            jax.ShapeDtypeStruct((N, 1), jnp.float32),
        ],
    )(x, dp)


def _convA_body(mp0_ref, mp1_ref, nd_ref, w_ref, x_ref, st_ref):
    i = pl.program_id(0)
    m0 = mp0_ref[0] + mp0_ref[1]
    m1 = mp1_ref[0] + mp1_ref[1]
    x = jnp.where(i < NB // 2, m0, m1) * nd_ref[...]
    x = jnp.dot(x, w_ref[...], preferred_element_type=jnp.float32)
    x_ref[...] = x

    @pl.when(i == 0)
    def _():
        st_ref[...] = jnp.zeros_like(st_ref)

    st_ref[0:1, :] += jnp.sum(x, axis=0, keepdims=True)
    st_ref[1:2, :] += jnp.sum(x * x, axis=0, keepdims=True)


def _convA(mp0, mp1, nd, w):
    return pl.pallas_call(
        _convA_body,
        grid=(NB,),
        in_specs=[
            pl.BlockSpec((NC, BN, D),
                         lambda i: (0, jnp.minimum(i, NB // 2 - 1), 0)),
            pl.BlockSpec((NC, BN, D),
                         lambda i: (0, jnp.maximum(i - NB // 2, 0), 0)),
            pl.BlockSpec((BN, 1), lambda i: (i, 0)),
            pl.BlockSpec((D, D), lambda i: (0, 0)),
        ],
        out_specs=[
            pl.BlockSpec((BN, D), lambda i: (i, 0)),
            pl.BlockSpec((2, D), lambda i: (0, 0)),
        ],
        out_shape=[
            jax.ShapeDtypeStruct((N, D), jnp.float32),
            jax.ShapeDtypeStruct((2, D), jnp.float32),
        ],
    )(mp0, mp1, nd, w)


def _convB_body(first, last, x_ref, st_ref, ga_ref, gg_ref, gb_ref,
                w1_ref, b1_ref, pr_ref, ns_ref, *out_refs):
    i = pl.program_id(0)
    a = ga_ref[...]
    mean = st_ref[0:1, :] * (1.0 / N)
    ex2 = st_ref[1:2, :] * (1.0 / N)
    var = ex2 - mean * mean * (2.0 * a - a * a)
    scale = gg_ref[...] * lax.rsqrt(var + EPS)
    uf = _leaky((x_ref[...] - a * mean) * scale + gb_ref[...])
    if first:
        ro_in = uf
        hnext = (uf + pr_ref[...]) * ns_ref[...]
    else:
        ro_in = uf + pr_ref[...]
        hnext = ro_in * ns_ref[...]
    hh = _leaky(jnp.dot(ro_in, w1_ref[...],
                        preferred_element_type=jnp.float32) + b1_ref[...])
    if last:
        s_ref, = out_refs
    else:
        h_ref, p_ref, s_ref = out_refs
        h_ref[...] = hnext
        p_ref[...] = uf

    @pl.when(i == 0)
    def _():
        s_ref[...] = jnp.zeros_like(s_ref)

    s_ref[...] += jnp.sum(hh, axis=0, keepdims=True)


def _convB(first, last, x, st, ga, gg, gb, w1, b1, pr, ns):
    out_specs = [pl.BlockSpec((1, D), lambda i: (0, 0))]
    out_shape = [jax.ShapeDtypeStruct((1, D), jnp.float32)]
    if not last:
        out_specs = [
            pl.BlockSpec((BN, D), lambda i: (i, 0)),
            pl.BlockSpec((BN, D), lambda i: (i, 0)),
        ] + out_specs
        out_shape = [
            jax.ShapeDtypeStruct((N, D), jnp.float32),
            jax.ShapeDtypeStruct((N, D), jnp.float32),
        ] + out_shape
    return pl.pallas_call(
        functools.partial(_convB_body, first, last),
        grid=(NB,),
        in_specs=[
            pl.BlockSpec((BN, D), lambda i: (i, 0)),
            pl.BlockSpec((2, D), lambda i: (0, 0)),
            pl.BlockSpec((1, D), lambda i: (0, 0)),
            pl.BlockSpec((1, D), lambda i: (0, 0)),
            pl.BlockSpec((1, D), lambda i: (0, 0)),
            pl.BlockSpec((D, D), lambda i: (0, 0)),
            pl.BlockSpec((1, D), lambda i: (0, 0)),
            pl.BlockSpec((BN, D), lambda i: (i, 0)),
            pl.BlockSpec((BN, 1), lambda i: (i, 0)),
        ],
        out_specs=out_specs,
        out_shape=out_shape,
    )(x, st, ga, gg, gb, w1, b1, pr, ns)


def _fin_body(s_ref, w2_ref, b2_ref, o_ref):
    for l in range(3):
        o_ref[l:l + 1, :] = _leaky(
            jnp.dot(s_ref[l:l + 1, :], w2_ref[l],
                    preferred_element_type=jnp.float32)
            + b2_ref[l])


def _fin(s, w2, b2):
    return pl.pallas_call(
        _fin_body,
        out_shape=jax.ShapeDtypeStruct((3, D), jnp.float32),
    )(s, w2, b2)


def kernel(node_feats, edge_index, W, gn_alpha, gn_gamma, gn_beta,
           ro_W1, ro_b1, ro_W2, ro_b2):
    npadc = NCHP - NCHUNKS
    junk = (N + jnp.arange(npadc * CHUNK, dtype=jnp.int32) % (NPAD - N)
            ).reshape(npadc, CHUNK)
    zpad = jnp.zeros((npadc, CHUNK), jnp.int32)
    srcj = jnp.concatenate([edge_index[0].reshape(NCHUNKS, CHUNK), junk])
    srcz = jnp.concatenate([edge_index[0].reshape(NCHUNKS, CHUNK), zpad])
    dst = edge_index[1].reshape(NCHUNKS, CHUNK)
    dst2 = jnp.concatenate([dst, junk])
    junkt = (NPH + jnp.arange(npadc * CHUNK, dtype=jnp.int32) % 256
             ).reshape(npadc, CHUNK)
    trash = NPH + (dst & 255)
    dstp0 = jnp.concatenate([jnp.where(dst < NPH, dst, trash), junkt])
    dstp1 = jnp.concatenate([jnp.where(dst >= NPH, dst - NPH, trash), junkt])
    deg_parts = _deg(srcj, dst2)
    dp = deg_parts.reshape(NC * 2, NPAD).T
    h, ns, nd = _pre(node_feats, dp)

    ga = gn_alpha.reshape(3, 1, D)
    gg = gn_gamma.reshape(3, 1, D)
    gb = gn_beta.reshape(3, 1, D)
    b1 = ro_b1.reshape(3, 1, D)

    ss = []
    prev_resid = node_feats
    for l in range(3):
        mp0 = _segsum(h, srcz, dstp0)
        mp1 = _segsum(h, srcz, dstp1)
        x, st = _convA(mp0, mp1, nd, W[l])
        first, last = l == 0, l == 2
        outs = _convB(first, last, x, st, ga[l], gg[l], gb[l],
                      ro_W1[l], b1[l], prev_resid, ns)
        if last:
            s, = outs
        else:
            h, prev_resid, s = outs
        ss.append(s)

    ro = _fin(jnp.concatenate(ss, axis=0), ro_W2, ro_b2.reshape(3, 1, D))
    return ro.reshape(3 * D)


# speedup vs baseline: 1.7834x; 1.7834x over previous
import functools

import jax
import jax.numpy as jnp
from jax import lax
from jax.experimental import pallas as pl
from jax.experimental.pallas import tpu as pltpu
from jax.experimental.pallas import tpu_sc as plsc

N = 10000
E = 320000
D = 128
EPS = 1e-5

NC = 2
NS = 16
NW = NC * NS
CHUNK = 64
NCHUNKS = E // CHUNK
G = 16
NGRP = 10
NCHP = 5120
WCH = NCHP // NW
NBUF = 4
NPAD = 10240
TROWS = NPAD // NS
ZROWS = 128
ZB = 32

BN = 1000
NB = N // BN

_mesh = plsc.VectorSubcoreMesh(core_axis_name="c", subcore_axis_name="s")


def _leaky(x):
    return jnp.where(x >= 0, x, 0.01 * x)


@functools.partial(
    pl.kernel,
    out_type=jax.ShapeDtypeStruct((NC, 2, NPAD), jnp.float32),
    mesh=_mesh,
    scratch_types=[
        pltpu.VMEM((G, CHUNK), jnp.int32),
        pltpu.VMEM((G, CHUNK), jnp.int32),
        pltpu.VMEM((G, CHUNK), jnp.float32),
        pltpu.VMEM((TROWS,), jnp.float32),
        pltpu.VMEM_SHARED((NPAD,), jnp.float32),
        pltpu.VMEM_SHARED((NPAD,), jnp.float32),
    ],
)
def _deg(src_hbm, dst_hbm, out_hbm, sbuf, dbuf, ones_v, zvec, acc_s, acc_d):
    cid = lax.axis_index("c")
    sid = lax.axis_index("s")
    w = sid * NC + cid

    def fill_ones(j, carry):
        for k in range(CHUNK // 16):
            ones_v[j, pl.ds(k * 16, 16)] = jnp.ones((16,), jnp.float32)
        return carry

    lax.fori_loop(0, G, fill_ones, 0)

    def fill_z(q, carry):
        zvec[pl.ds(q * 16, 16)] = jnp.zeros((16,), jnp.float32)
        return carry

    lax.fori_loop(0, TROWS // 16, fill_z, 0)
    pltpu.sync_copy(zvec, acc_s.at[pl.ds(sid * TROWS, TROWS)])
    pltpu.sync_copy(zvec, acc_d.at[pl.ds(sid * TROWS, TROWS)])
    plsc.subcore_barrier()

    cstart = w * WCH

    def grp(g, carry):
        gs = cstart + g * G
        pltpu.sync_copy(src_hbm.at[pl.ds(gs, G)], sbuf)
        pltpu.sync_copy(dst_hbm.at[pl.ds(gs, G)], dbuf)

        def chunk(j, c2):
            pltpu.sync_copy(ones_v.at[0], acc_s.at[sbuf.at[j]], add=True)
            pltpu.sync_copy(ones_v.at[0], acc_d.at[dbuf.at[j]], add=True)
            return c2

        lax.fori_loop(0, G, chunk, 0)
        return carry

    lax.fori_loop(0, NGRP, grp, 0)
    plsc.subcore_barrier()
    pltpu.sync_copy(acc_s.at[pl.ds(sid * TROWS, TROWS)],
                    out_hbm.at[cid, 0, pl.ds(sid * TROWS, TROWS)])
    pltpu.sync_copy(acc_d.at[pl.ds(sid * TROWS, TROWS)],
                    out_hbm.at[cid, 1, pl.ds(sid * TROWS, TROWS)])


@functools.partial(
    pl.kernel,
    out_type=jax.ShapeDtypeStruct((NC, NPAD, D), jnp.float32),
    mesh=_mesh,
    scratch_types=[
        pltpu.VMEM((2, G, CHUNK), jnp.int32),
        pltpu.VMEM((2, G, CHUNK), jnp.int32),
        pltpu.VMEM((CHUNK, D), jnp.float32),
        pltpu.VMEM((CHUNK, D), jnp.float32),
        pltpu.VMEM((CHUNK, D), jnp.float32),
        pltpu.VMEM((CHUNK, D), jnp.float32),
        pltpu.VMEM((ZB, D), jnp.float32),
        pltpu.VMEM_SHARED((NPAD, D), jnp.float32),
        pltpu.SemaphoreType.DMA,
        pltpu.SemaphoreType.DMA,
        pltpu.SemaphoreType.DMA,
        pltpu.SemaphoreType.DMA,
        pltpu.SemaphoreType.DMA,
        pltpu.SemaphoreType.DMA,
        pltpu.SemaphoreType.DMA,
        pltpu.SemaphoreType.DMA,
    ],
)
def _segsum(h_hbm, src_hbm, dst_hbm, out_hbm,
            sbuf, dbuf, rows0, rows1, rows2, rows3, zbuf, acc,
            gs0, gs1, gs2, gs3, ss0, ss1, ss2, ss3):
    rows = (rows0, rows1, rows2, rows3)
    gsem = (gs0, gs1, gs2, gs3)
    ssem = (ss0, ss1, ss2, ss3)
    cid = lax.axis_index("c")
    sid = lax.axis_index("s")
    w = sid * NC + cid

    def zrow(i, carry):
        for k in range(D // 16):
            zbuf[i, pl.ds(k * 16, 16)] = jnp.zeros((16,), jnp.float32)
        return carry

    lax.fori_loop(0, ZB, zrow, 0)
    for b in range(TROWS // ZB):
        pltpu.sync_copy(zbuf, acc.at[pl.ds(sid * TROWS + b * ZB, ZB)])
    plsc.subcore_barrier()

    cstart = w * WCH

    def grp(g, carry):
        p = g % 2
        gs = cstart + g * G
        pltpu.sync_copy(src_hbm.at[pl.ds(gs, G)], sbuf.at[p])
        pltpu.sync_copy(dst_hbm.at[pl.ds(gs, G)], dbuf.at[p])

        def quad(kk, c2):
            for b in range(NBUF):
                j = kk * NBUF + b
                c = g * G + j

                @pl.when(c >= NBUF)
                def _():
                    pltpu.make_async_copy(
                        rows[b], acc.at[dbuf.at[p, j]], ssem[b]).wait()

                pltpu.async_copy(h_hbm.at[sbuf.at[p, j]], rows[b], gsem[b])

            for b in range(NBUF):
                j = kk * NBUF + b
                pltpu.make_async_copy(
                    h_hbm.at[sbuf.at[p, j]], rows[b], gsem[b]).wait()
                pltpu.async_copy(
                    rows[b], acc.at[dbuf.at[p, j]], ssem[b], add=True)

            return c2

        lax.fori_loop(0, G // NBUF, quad, 0)
        return carry

    lax.fori_loop(0, NGRP, grp, 0)
    for b in range(NBUF):
        pltpu.make_async_copy(rows[b], acc.at[dbuf.at[0, 0]], ssem[b]).wait()
    plsc.subcore_barrier()
    for b in range(TROWS // ZROWS):
        r0 = sid * TROWS + b * ZROWS
        pltpu.sync_copy(acc.at[pl.ds(r0, ZROWS)],
                        out_hbm.at[cid, pl.ds(r0, ZROWS)])


def _pre_body(x_ref, dp_ref, h0_ref, ns_ref, nd_ref):
    dp = dp_ref[...]
    ns = lax.rsqrt(jnp.maximum(dp[:, 0:1] + dp[:, 2:3], 1.0))
    nd = lax.rsqrt(jnp.maximum(dp[:, 1:2] + dp[:, 3:4], 1.0))
    ns_ref[...] = ns
    nd_ref[...] = nd
    h0_ref[...] = x_ref[...] * ns


def _pre(x, dp):
    return pl.pallas_call(
        _pre_body,
        grid=(NB,),
        in_specs=[
            pl.BlockSpec((BN, D), lambda i: (i, 0)),
            pl.BlockSpec((BN, 4), lambda i: (i, 0)),
        ],
        out_specs=[
            pl.BlockSpec((BN, D), lambda i: (i, 0)),
            pl.BlockSpec((BN, 1), lambda i: (i, 0)),
            pl.BlockSpec((BN, 1), lambda i: (i, 0)),
        ],
        out_shape=[
            jax.ShapeDtypeStruct((N, D), jnp.float32),
            jax.ShapeDtypeStruct((N, 1), jnp.float32),
            jax.ShapeDtypeStruct((N, 1), jnp.float32),
        ],
    )(x, dp)


def _convA_body(mp_ref, nd_ref, w_ref, x_ref, st_ref):
    i = pl.program_id(0)
    x = (mp_ref[0] + mp_ref[1]) * nd_ref[...]
    x = jnp.dot(x, w_ref[...], preferred_element_type=jnp.float32)
    x_ref[...] = x

    @pl.when(i == 0)
    def _():
        st_ref[...] = jnp.zeros_like(st_ref)

    st_ref[0:1, :] += jnp.sum(x, axis=0, keepdims=True)
    st_ref[1:2, :] += jnp.sum(x * x, axis=0, keepdims=True)


def _convA(mp, nd, w):
    return pl.pallas_call(
        _convA_body,
        grid=(NB,),
        in_specs=[
            pl.BlockSpec((NC, BN, D), lambda i: (0, i, 0)),
            pl.BlockSpec((BN, 1), lambda i: (i, 0)),
            pl.BlockSpec((D, D), lambda i: (0, 0)),
        ],
        out_specs=[
            pl.BlockSpec((BN, D), lambda i: (i, 0)),
            pl.BlockSpec((2, D), lambda i: (0, 0)),
        ],
        out_shape=[
            jax.ShapeDtypeStruct((N, D), jnp.float32),
            jax.ShapeDtypeStruct((2, D), jnp.float32),
        ],
    )(mp, nd, w)


def _convB_body(first, last, x_ref, st_ref, ga_ref, gg_ref, gb_ref,
                w1_ref, b1_ref, pr_ref, ns_ref, *out_refs):
    i = pl.program_id(0)
    a = ga_ref[...]
    mean = st_ref[0:1, :] * (1.0 / N)
    ex2 = st_ref[1:2, :] * (1.0 / N)
    var = ex2 - mean * mean * (2.0 * a - a * a)
    scale = gg_ref[...] * lax.rsqrt(var + EPS)
    uf = _leaky((x_ref[...] - a * mean) * scale + gb_ref[...])
    if first:
        ro_in = uf
        hnext = (uf + pr_ref[...]) * ns_ref[...]
    else:
        ro_in = uf + pr_ref[...]
        hnext = ro_in * ns_ref[...]
    hh = _leaky(jnp.dot(ro_in, w1_ref[...],
                        preferred_element_type=jnp.float32) + b1_ref[...])
    if last:
        s_ref, = out_refs
    else:
        h_ref, p_ref, s_ref = out_refs
        h_ref[...] = hnext
        p_ref[...] = uf

    @pl.when(i == 0)
    def _():
        s_ref[...] = jnp.zeros_like(s_ref)

    s_ref[...] += jnp.sum(hh, axis=0, keepdims=True)


def _convB(first, last, x, st, ga, gg, gb, w1, b1, pr, ns):
    out_specs = [pl.BlockSpec((1, D), lambda i: (0, 0))]
    out_shape = [jax.ShapeDtypeStruct((1, D), jnp.float32)]
    if not last:
        out_specs = [
            pl.BlockSpec((BN, D), lambda i: (i, 0)),
            pl.BlockSpec((BN, D), lambda i: (i, 0)),
        ] + out_specs
        out_shape = [
            jax.ShapeDtypeStruct((N, D), jnp.float32),
            jax.ShapeDtypeStruct((N, D), jnp.float32),
        ] + out_shape
    return pl.pallas_call(
        functools.partial(_convB_body, first, last),
        grid=(NB,),
        in_specs=[
            pl.BlockSpec((BN, D), lambda i: (i, 0)),
            pl.BlockSpec((2, D), lambda i: (0, 0)),
            pl.BlockSpec((1, D), lambda i: (0, 0)),
            pl.BlockSpec((1, D), lambda i: (0, 0)),
            pl.BlockSpec((1, D), lambda i: (0, 0)),
            pl.BlockSpec((D, D), lambda i: (0, 0)),
            pl.BlockSpec((1, D), lambda i: (0, 0)),
            pl.BlockSpec((BN, D), lambda i: (i, 0)),
            pl.BlockSpec((BN, 1), lambda i: (i, 0)),
        ],
        out_specs=out_specs,
        out_shape=out_shape,
    )(x, st, ga, gg, gb, w1, b1, pr, ns)


def _fin_body(s_ref, w2_ref, b2_ref, o_ref):
    for l in range(3):
        o_ref[l:l + 1, :] = _leaky(
            jnp.dot(s_ref[l:l + 1, :], w2_ref[l],
                    preferred_element_type=jnp.float32)
            + b2_ref[l])


def _fin(s, w2, b2):
    return pl.pallas_call(
        _fin_body,
        out_shape=jax.ShapeDtypeStruct((3, D), jnp.float32),
    )(s, w2, b2)


def kernel(node_feats, edge_index, W, gn_alpha, gn_gamma, gn_beta,
           ro_W1, ro_b1, ro_W2, ro_b2):
    npadc = NCHP - NCHUNKS
    junk = (N + jnp.arange(npadc * CHUNK, dtype=jnp.int32) % (NPAD - N)
            ).reshape(npadc, CHUNK)
    zpad = jnp.zeros((npadc, CHUNK), jnp.int32)
    srcj = jnp.concatenate([edge_index[0].reshape(NCHUNKS, CHUNK), junk])
    srcz = jnp.concatenate([edge_index[0].reshape(NCHUNKS, CHUNK), zpad])
    dst2 = jnp.concatenate([edge_index[1].reshape(NCHUNKS, CHUNK), junk])
    deg_parts = _deg(srcj, dst2)
    dp = deg_parts.reshape(NC * 2, NPAD).T
    h, ns, nd = _pre(node_feats, dp)

    ga = gn_alpha.reshape(3, 1, D)
    gg = gn_gamma.reshape(3, 1, D)
    gb = gn_beta.reshape(3, 1, D)
    b1 = ro_b1.reshape(3, 1, D)

    ss = []
    prev_resid = node_feats
    for l in range(3):
        mp = _segsum(h, srcz, dst2)
        x, st = _convA(mp, nd, W[l])
        first, last = l == 0, l == 2
        outs = _convB(first, last, x, st, ga[l], gg[l], gb[l],
                      ro_W1[l], b1[l], prev_resid, ns)
        if last:
            s, = outs
        else:
            h, prev_resid, s = outs
        ss.append(s)

    ro = _fin(jnp.concatenate(ss, axis=0), ro_W2, ro_b2.reshape(3, 1, D))
    return ro.reshape(3 * D)
